# baseline (speedup 1.0000x reference)
import functools

import jax
import jax.numpy as jnp
from jax import lax
from jax.experimental import pallas as pl
from jax.experimental.pallas import tpu as pltpu
from jax.experimental.pallas import tpu_sc as plsc

N = 10000
E = 320000
D = 128
H = 128
ED = 16

NC = 2
NS = 16
NW = NC * NS
K = 40
EPT = E // NW
CHUNKS = EPT // K
IBLK = 10
NBLK = CHUNKS // IBLK
NP = 10112
RPT = NP // NS
LANES = 16
HW = H // 2

_DOT = functools.partial(
    lax.dot_general,
    dimension_numbers=(((1,), (0,)), ((), ())),
    preferred_element_type=jnp.float32,
    precision=lax.Precision.HIGHEST,
)


def _pack_bf16_pair(v):
    i = lax.bitcast_convert_type(v, jnp.int32)
    rbit = jnp.bitwise_and(lax.shift_right_logical(i, 16), 1)
    r = jnp.bitwise_and(
        lax.shift_right_arithmetic(i + 0x7FFF + rbit, 16), 0xFFFF)
    return jnp.bitwise_or(r[:, :HW], lax.shift_left(r[:, HW:], 16))


def _pre_body(x_ref, wi_ref, wj_ref, a_ref, b_ref):
    xb = x_ref[...]
    a_ref[...] = _DOT(xb, wi_ref[...])
    b_ref[...] = _DOT(xb, wj_ref[...])


def _cpre_body(ea_ref, we_ref, b_ref, c_ref):
    c_ref[...] = _pack_bf16_pair(_DOT(ea_ref[...], we_ref[...]) + b_ref[...])


def _upd_fused_body(x_ref, a0_ref, a1_ref, wux_ref, wua_ref, bu_ref,
                    wi_ref, wj_ref, h_ref, a_ref, b_ref):
    agg = a0_ref[0] + a1_ref[0]
    h = _DOT(x_ref[...], wux_ref[...]) + _DOT(agg, wua_ref[...]) + bu_ref[...]
    h = jnp.maximum(h, 0.0)
    h_ref[...] = h
    a_ref[...] = _DOT(h, wi_ref[...])
    b_ref[...] = _DOT(h, wj_ref[...])


def _upd_body(x_ref, a0_ref, a1_ref, wux_ref, wua_ref, bu_ref, h_ref):
    agg = a0_ref[0] + a1_ref[0]
    h = _DOT(x_ref[...], wux_ref[...]) + _DOT(agg, wua_ref[...]) + bu_ref[...]
    h_ref[...] = jnp.maximum(h, 0.0)


_BN = 1000
_BE = 4000


def _node_spec(shape):
    return pl.BlockSpec((_BN,) + shape[1:], lambda i: (i,) + (0,) * (len(shape) - 1))


def _full_spec(shape):
    return pl.BlockSpec(shape, lambda i: (0,) * len(shape))


def _tc_pre(x, wi, wj):
    return pl.pallas_call(
        _pre_body,
        grid=(N // _BN,),
        in_specs=[_node_spec((N, D)), _full_spec((D, H)), _full_spec((D, H))],
        out_specs=[_node_spec((N, H)), _node_spec((N, H))],
        out_shape=[jax.ShapeDtypeStruct((N, H), jnp.float32)] * 2,
    )(x, wi, wj)


def _tc_cpre(ea, we, b):
    espec = pl.BlockSpec((_BE, ED), lambda i: (i, 0))
    ospec = pl.BlockSpec((_BE, HW), lambda i: (i, 0))
    return pl.pallas_call(
        _cpre_body,
        grid=(E // _BE,),
        in_specs=[espec, _full_spec((ED, H)), _full_spec((1, H))],
        out_specs=ospec,
        out_shape=jax.ShapeDtypeStruct((E, HW), jnp.int32),
    )(ea, we, b)


_A0SPEC = pl.BlockSpec((1, _BN, H), lambda i: (0, i, 0))
_A1SPEC = pl.BlockSpec((1, _BN, H), lambda i: (1, i, 0))


def _tc_update_fused(x, aggs, wux, wua, bu, wi, wj):
    return pl.pallas_call(
        _upd_fused_body,
        grid=(N // _BN,),
        in_specs=[_node_spec((N, D)), _A0SPEC, _A1SPEC,
                  _full_spec((D, H)), _full_spec((H, H)), _full_spec((1, H)),
                  _full_spec((H, H)), _full_spec((H, H))],
        out_specs=[_node_spec((N, H))] * 3,
        out_shape=[jax.ShapeDtypeStruct((N, H), jnp.float32)] * 3,
    )(x, aggs, aggs, wux, wua, bu, wi, wj)


def _tc_update(x, aggs, wux, wua, bu):
    return pl.pallas_call(
        _upd_body,
        grid=(N // _BN,),
        in_specs=[_node_spec((N, H)), _A0SPEC, _A1SPEC,
                  _full_spec((H, H)), _full_spec((H, H)), _full_spec((1, H))],
        out_specs=_node_spec((N, H)),
        out_shape=jax.ShapeDtypeStruct((N, H), jnp.float32),
    )(x, aggs, aggs, wux, wua, bu)


def _unpack_pair(w):
    lo = lax.bitcast_convert_type(lax.shift_left(w, 16), jnp.float32)
    hi = lax.bitcast_convert_type(
        jnp.bitwise_and(w, jnp.int32(-65536)), jnp.float32)
    return lo, hi


def _sc_edge_body(a_hbm, b_hbm, c_hbm, src_hbm, dst_hbm, out_hbm,
                  shared, idx_d0, idx_d1, idx_s0, idx_s1,
                  buf_a0, buf_a1, buf_b0, buf_b1, buf_c0, buf_c1,
                  msg0, msg1,
                  sem_a0, sem_a1, sem_b0, sem_b1, sem_c0, sem_c1,
                  sem_s0, sem_s1, sem_id0, sem_id1, sem_is0, sem_is1):
    c = lax.axis_index("c")
    s = lax.axis_index("s")
    g = c * NS + s

    idx_d = (idx_d0, idx_d1)
    idx_s = (idx_s0, idx_s1)
    buf_a = (buf_a0, buf_a1)
    buf_b = (buf_b0, buf_b1)
    buf_c = (buf_c0, buf_c1)
    msg = (msg0, msg1)
    sem_a = (sem_a0, sem_a1)
    sem_b = (sem_b0, sem_b1)
    sem_c = (sem_c0, sem_c1)
    sem_s = (sem_s0, sem_s1)
    sem_id = (sem_id0, sem_id1)
    sem_is = (sem_is0, sem_is1)

    def _load_idx(b, q):
        pltpu.async_copy(dst_hbm.at[g, b], idx_d[q], sem_id[q])
        pltpu.async_copy(src_hbm.at[g, b], idx_s[q], sem_is[q])

    def _wait_idx(q):
        pltpu.make_async_copy(dst_hbm.at[0, 0], idx_d[q], sem_id[q]).wait()
        pltpu.make_async_copy(src_hbm.at[0, 0], idx_s[q], sem_is[q]).wait()

    _load_idx(0, 0)

    zero = jnp.zeros((LANES,), jnp.float32)

    def _zrow(r, _):
        for j in range(H // LANES):
            msg0[r, pl.ds(j * LANES, LANES)] = zero
        return 0
    lax.fori_loop(0, K, _zrow, 0, unroll=False)

    rbase = pl.multiple_of(s * RPT, 8)
    nfull = RPT // K
    rem = RPT - nfull * K
    for j in range(nfull):
        pltpu.sync_copy(msg0, shared.at[pl.ds(rbase + j * K, K)])
    if rem:
        pltpu.sync_copy(msg0.at[pl.ds(0, rem)],
                        shared.at[pl.ds(rbase + nfull * K, rem)])

    plsc.subcore_barrier()

    def _gather_ab(blk, j, p, q):
        pltpu.async_copy(a_hbm.at[idx_d[q].at[j]], buf_a[p], sem_a[p])
        pltpu.async_copy(b_hbm.at[idx_s[q].at[j]], buf_b[p], sem_b[p])

    def _load_c(blk, j, p):
        eoff = pl.multiple_of(g * EPT + (blk * IBLK + j) * K, 8)
        pltpu.async_copy(c_hbm.at[pl.ds(eoff, K)], buf_c[p], sem_c[p])

    def _step(blk, j, p, q):
        pltpu.make_async_copy(a_hbm.at[pl.ds(0, K)], buf_a[p], sem_a[p]).wait()
        pltpu.make_async_copy(b_hbm.at[pl.ds(0, K)], buf_b[p], sem_b[p]).wait()
        pltpu.make_async_copy(c_hbm.at[pl.ds(0, K)], buf_c[p], sem_c[p]).wait()

        def _row(r, _):
            for jj in range(HW // LANES):
                lo_sl = pl.ds(jj * LANES, LANES)
                hi_sl = pl.ds(HW + jj * LANES, LANES)
                clo, chi = _unpack_pair(buf_c[p][r, lo_sl])
                msg[p][r, lo_sl] = jnp.maximum(
                    buf_a[p][r, lo_sl] + buf_b[p][r, lo_sl] + clo, 0.0)
                msg[p][r, hi_sl] = jnp.maximum(
                    buf_a[p][r, hi_sl] + buf_b[p][r, hi_sl] + chi, 0.0)
            return 0
        lax.fori_loop(0, K, _row, 0, unroll=False)

        pltpu.async_copy(msg[p], shared.at[idx_d[q].at[j]], sem_s[p],
                         add=True)
        nxt = j + 2

        @pl.when(nxt < IBLK)
        def _():
            _gather_ab(blk, nxt, p, q)
            _load_c(blk, nxt, p)

        pltpu.make_async_copy(msg[p], shared.at[pl.ds(0, K)], sem_s[p]).wait()

    for blk in range(NBLK):
        q = blk % 2
        _wait_idx(q)
        if blk + 1 < NBLK:
            _load_idx(blk + 1, 1 - q)
        _gather_ab(blk, 0, 0, q)
        _load_c(blk, 0, 0)
        _gather_ab(blk, 1, 1, q)
        _load_c(blk, 1, 1)

        def _super(t, _):
            _step(blk, 2 * t, 0, q)
            _step(blk, 2 * t + 1, 1, q)
            return 0
        lax.fori_loop(0, IBLK // 2, _super, 0, unroll=False)
        if IBLK % 2:
            _step(blk, IBLK - 1, 0, q)

    plsc.subcore_barrier()

    obase = pl.multiple_of(c * NP + rbase, 8)
    for j in range(nfull):
        pltpu.sync_copy(shared.at[pl.ds(rbase + j * K, K)], msg0)
        pltpu.sync_copy(msg0, out_hbm.at[pl.ds(obase + j * K, K)])
    if rem:
        pltpu.sync_copy(shared.at[pl.ds(rbase + nfull * K, rem)],
                        msg0.at[pl.ds(0, rem)])
        pltpu.sync_copy(msg0.at[pl.ds(0, rem)],
                        out_hbm.at[pl.ds(obase + nfull * K, rem)])


@functools.cache
def _sc_edge_kernel():
    return pl.kernel(
        _sc_edge_body,
        out_type=jax.ShapeDtypeStruct((NC * NP, H), jnp.float32),
        mesh=plsc.VectorSubcoreMesh(core_axis_name="c", subcore_axis_name="s"),
        scratch_types=[
            pltpu.VMEM_SHARED((NP, H), jnp.float32),
        ] + [pltpu.VMEM((IBLK, K), jnp.int32)] * 4
          + [pltpu.VMEM((K, H), jnp.float32)] * 4
          + [pltpu.VMEM((K, HW), jnp.int32)] * 2
          + [pltpu.VMEM((K, H), jnp.float32)] * 2
          + [pltpu.SemaphoreType.DMA] * 12,
    )


def _sc_edge(a, b, c, src, dst):
    src4 = src.reshape(NW, NBLK, IBLK, K)
    dst4 = dst.reshape(NW, NBLK, IBLK, K)
    return _sc_edge_kernel()(a, b, c, src4, dst4)


@jax.jit
def kernel(x, edge_index, edge_attr, W_msg0, b_msg0, W_upd0, b_upd0,
           W_msg1, b_msg1, W_upd1, b_upd1):
    src = edge_index[0]
    dst = edge_index[1]

    b0 = b_msg0.reshape(1, H)
    b1 = b_msg1.reshape(1, H)
    bu0 = b_upd0.reshape(1, H)
    bu1 = b_upd1.reshape(1, H)

    c0 = _tc_cpre(edge_attr, W_msg0[2 * D:], b0)
    a0, bmat0 = _tc_pre(x, W_msg0[:D], W_msg0[D:2 * D])
    aggs0 = _sc_edge(a0, bmat0, c0, src, dst).reshape(NC, NP, H)
    c1 = _tc_cpre(edge_attr, W_msg1[2 * H:], b1)
    h, a1, bmat1 = _tc_update_fused(
        x, aggs0, W_upd0[:D], W_upd0[D:], bu0,
        W_msg1[:H], W_msg1[H:2 * H])

    aggs1 = _sc_edge(a1, bmat1, c1, src, dst).reshape(NC, NP, H)
    out = _tc_update(h, aggs1, W_upd1[:H], W_upd1[H:], bu1)
    return out


# speedup vs baseline: 1.0492x; 1.0492x over previous
import functools

import jax
import jax.numpy as jnp
from jax import lax
from jax.experimental import pallas as pl
from jax.experimental.pallas import tpu as pltpu
from jax.experimental.pallas import tpu_sc as plsc

N = 10000
E = 320000
D = 128
H = 128
ED = 16

NC = 2
NS = 16
NW = NC * NS
K = 40
EPT = E // NW
CHUNKS = EPT // K
IBLK = 25
NBLK = CHUNKS // IBLK
NP = 10112
RPT = NP // NS
LANES = 16
HW = H // 2

_DOT = functools.partial(
    lax.dot_general,
    dimension_numbers=(((1,), (0,)), ((), ())),
    preferred_element_type=jnp.float32,
    precision=lax.Precision.HIGHEST,
)


def _pack_bf16_pair(v):
    i = lax.bitcast_convert_type(v, jnp.int32)
    rbit = jnp.bitwise_and(lax.shift_right_logical(i, 16), 1)
    r = jnp.bitwise_and(
        lax.shift_right_arithmetic(i + 0x7FFF + rbit, 16), 0xFFFF)
    return jnp.bitwise_or(r[:, :HW], lax.shift_left(r[:, HW:], 16))


def _pre_body(x_ref, wi_ref, wj_ref, a_ref, b_ref):
    xb = x_ref[...]
    a_ref[...] = _DOT(xb, wi_ref[...])
    b_ref[...] = _DOT(xb, wj_ref[...])


def _cpre_body(ea_ref, we_ref, b_ref, c_ref):
    c_ref[...] = _DOT(ea_ref[...], we_ref[...]) + b_ref[...]


def _upd_fused_body(x_ref, a0_ref, a1_ref, wux_ref, wua_ref, bu_ref,
                    wi_ref, wj_ref, h_ref, a_ref, b_ref):
    agg = a0_ref[0] + a1_ref[0]
    h = _DOT(x_ref[...], wux_ref[...]) + _DOT(agg, wua_ref[...]) + bu_ref[...]
    h = jnp.maximum(h, 0.0)
    h_ref[...] = h
    a_ref[...] = _DOT(h, wi_ref[...])
    b_ref[...] = _DOT(h, wj_ref[...])


def _upd_body(x_ref, a0_ref, a1_ref, wux_ref, wua_ref, bu_ref, h_ref):
    agg = a0_ref[0] + a1_ref[0]
    h = _DOT(x_ref[...], wux_ref[...]) + _DOT(agg, wua_ref[...]) + bu_ref[...]
    h_ref[...] = jnp.maximum(h, 0.0)


_BN = 1000
_BE = 4000


def _node_spec(shape):
    return pl.BlockSpec((_BN,) + shape[1:], lambda i: (i,) + (0,) * (len(shape) - 1))


def _full_spec(shape):
    return pl.BlockSpec(shape, lambda i: (0,) * len(shape))


def _tc_pre(x, wi, wj):
    return pl.pallas_call(
        _pre_body,
        grid=(N // _BN,),
        in_specs=[_node_spec((N, D)), _full_spec((D, H)), _full_spec((D, H))],
        out_specs=[_node_spec((N, H)), _node_spec((N, H))],
        out_shape=[jax.ShapeDtypeStruct((N, H), jnp.float32)] * 2,
    )(x, wi, wj)


def _tc_cpre(ea, we, b):
    espec = pl.BlockSpec((_BE, ED), lambda i: (i, 0))
    ospec = pl.BlockSpec((_BE, H), lambda i: (i, 0))
    return pl.pallas_call(
        _cpre_body,
        grid=(E // _BE,),
        in_specs=[espec, _full_spec((ED, H)), _full_spec((1, H))],
        out_specs=ospec,
        out_shape=jax.ShapeDtypeStruct((E, H), jnp.float32),
    )(ea, we, b)


_A0SPEC = pl.BlockSpec((1, _BN, H), lambda i: (0, i, 0))
_A1SPEC = pl.BlockSpec((1, _BN, H), lambda i: (1, i, 0))


def _tc_update_fused(x, aggs, wux, wua, bu, wi, wj):
    return pl.pallas_call(
        _upd_fused_body,
        grid=(N // _BN,),
        in_specs=[_node_spec((N, D)), _A0SPEC, _A1SPEC,
                  _full_spec((D, H)), _full_spec((H, H)), _full_spec((1, H)),
                  _full_spec((H, H)), _full_spec((H, H))],
        out_specs=[_node_spec((N, H))] * 3,
        out_shape=[jax.ShapeDtypeStruct((N, H), jnp.float32)] * 3,
    )(x, aggs, aggs, wux, wua, bu, wi, wj)


def _tc_update(x, aggs, wux, wua, bu):
    return pl.pallas_call(
        _upd_body,
        grid=(N // _BN,),
        in_specs=[_node_spec((N, H)), _A0SPEC, _A1SPEC,
                  _full_spec((H, H)), _full_spec((H, H)), _full_spec((1, H))],
        out_specs=_node_spec((N, H)),
        out_shape=jax.ShapeDtypeStruct((N, H), jnp.float32),
    )(x, aggs, aggs, wux, wua, bu)


def _unpack_pair(w):
    lo = lax.bitcast_convert_type(lax.shift_left(w, 16), jnp.float32)
    hi = lax.bitcast_convert_type(
        jnp.bitwise_and(w, jnp.int32(-65536)), jnp.float32)
    return lo, hi


def _sc_edge_body(a_hbm, b_hbm, c_hbm, src_hbm, dst_hbm, out_hbm,
                  shared, idx_d0, idx_d1, idx_s0, idx_s1,
                  buf_a0, buf_a1, buf_b0, buf_b1, buf_c0, buf_c1,
                  sem_a0, sem_a1, sem_b0, sem_b1, sem_c0, sem_c1,
                  sem_s0, sem_s1, sem_id0, sem_id1, sem_is0, sem_is1):
    c = lax.axis_index("c")
    s = lax.axis_index("s")
    g = c * NS + s

    idx_d = (idx_d0, idx_d1)
    idx_s = (idx_s0, idx_s1)
    buf_a = (buf_a0, buf_a1)
    buf_b = (buf_b0, buf_b1)
    buf_c = (buf_c0, buf_c1)
    sem_a = (sem_a0, sem_a1)
    sem_b = (sem_b0, sem_b1)
    sem_c = (sem_c0, sem_c1)
    sem_s = (sem_s0, sem_s1)
    sem_id = (sem_id0, sem_id1)
    sem_is = (sem_is0, sem_is1)

    def _load_idx(b, q):
        pltpu.async_copy(dst_hbm.at[g, b], idx_d[q], sem_id[q])
        pltpu.async_copy(src_hbm.at[g, b], idx_s[q], sem_is[q])

    def _wait_idx(q):
        pltpu.make_async_copy(dst_hbm.at[0, 0], idx_d[q], sem_id[q]).wait()
        pltpu.make_async_copy(src_hbm.at[0, 0], idx_s[q], sem_is[q]).wait()

    _load_idx(0, 0)

    zero = jnp.zeros((LANES,), jnp.float32)

    def _zrow(r, _):
        for j in range(H // LANES):
            buf_a0[r, pl.ds(j * LANES, LANES)] = zero
        return 0
    lax.fori_loop(0, K, _zrow, 0, unroll=False)

    rbase = pl.multiple_of(s * RPT, 8)
    nfull = RPT // K
    rem = RPT - nfull * K
    for j in range(nfull):
        pltpu.sync_copy(buf_a0, shared.at[pl.ds(rbase + j * K, K)])
    if rem:
        pltpu.sync_copy(buf_a0.at[pl.ds(0, rem)],
                        shared.at[pl.ds(rbase + nfull * K, rem)])

    plsc.subcore_barrier()

    def _gather_ab(blk, j, p, q):
        pltpu.async_copy(a_hbm.at[idx_d[q].at[j]], buf_a[p], sem_a[p])
        pltpu.async_copy(b_hbm.at[idx_s[q].at[j]], buf_b[p], sem_b[p])

    def _load_c(blk, j, p):
        eoff = pl.multiple_of(g * EPT + (blk * IBLK + j) * K, 8)
        pltpu.async_copy(c_hbm.at[pl.ds(eoff, K)], buf_c[p], sem_c[p])

    def _step(blk, j, p, q):
        pltpu.make_async_copy(a_hbm.at[pl.ds(0, K)], buf_a[p], sem_a[p]).wait()
        pltpu.make_async_copy(b_hbm.at[pl.ds(0, K)], buf_b[p], sem_b[p]).wait()
        pltpu.make_async_copy(c_hbm.at[pl.ds(0, K)], buf_c[p], sem_c[p]).wait()

        def _row(r, _):
            for jj in range(H // LANES):
                sl = pl.ds(jj * LANES, LANES)
                v = buf_a[p][r, sl] + buf_b[p][r, sl] + buf_c[p][r, sl]
                buf_c[p][r, sl] = jnp.maximum(v, 0.0)
            return 0
        lax.fori_loop(0, K, _row, 0, unroll=False)

        pltpu.async_copy(buf_c[p], shared.at[idx_d[q].at[j]], sem_s[p],
                         add=True)
        nxt = j + 2

        @pl.when(nxt < IBLK)
        def _():
            _gather_ab(blk, nxt, p, q)

        pltpu.make_async_copy(buf_c[p], shared.at[pl.ds(0, K)], sem_s[p]).wait()

        @pl.when(nxt < IBLK)
        def _():
            _load_c(blk, nxt, p)

    for blk in range(NBLK):
        q = blk % 2
        _wait_idx(q)
        if blk + 1 < NBLK:
            _load_idx(blk + 1, 1 - q)
        _gather_ab(blk, 0, 0, q)
        _load_c(blk, 0, 0)
        _gather_ab(blk, 1, 1, q)
        _load_c(blk, 1, 1)

        def _super(t, _):
            _step(blk, 2 * t, 0, q)
            _step(blk, 2 * t + 1, 1, q)
            return 0
        lax.fori_loop(0, IBLK // 2, _super, 0, unroll=False)
        if IBLK % 2:
            _step(blk, IBLK - 1, 0, q)

    plsc.subcore_barrier()

    obase = pl.multiple_of(c * NP + rbase, 8)
    for j in range(nfull):
        pltpu.sync_copy(shared.at[pl.ds(rbase + j * K, K)], buf_a0)
        pltpu.sync_copy(buf_a0, out_hbm.at[pl.ds(obase + j * K, K)])
    if rem:
        pltpu.sync_copy(shared.at[pl.ds(rbase + nfull * K, rem)],
                        buf_a0.at[pl.ds(0, rem)])
        pltpu.sync_copy(buf_a0.at[pl.ds(0, rem)],
                        out_hbm.at[pl.ds(obase + nfull * K, rem)])


@functools.cache
def _sc_edge_kernel():
    return pl.kernel(
        _sc_edge_body,
        out_type=jax.ShapeDtypeStruct((NC * NP, H), jnp.float32),
        mesh=plsc.VectorSubcoreMesh(core_axis_name="c", subcore_axis_name="s"),
        scratch_types=[
            pltpu.VMEM_SHARED((NP, H), jnp.float32),
        ] + [pltpu.VMEM((IBLK, K), jnp.int32)] * 4
          + [pltpu.VMEM((K, H), jnp.float32)] * 6
          + [pltpu.SemaphoreType.DMA] * 12,
    )


def _sc_edge(a, b, c, src, dst):
    src4 = src.reshape(NW, NBLK, IBLK, K)
    dst4 = dst.reshape(NW, NBLK, IBLK, K)
    return _sc_edge_kernel()(a, b, c, src4, dst4)


@jax.jit
def kernel(x, edge_index, edge_attr, W_msg0, b_msg0, W_upd0, b_upd0,
           W_msg1, b_msg1, W_upd1, b_upd1):
    src = edge_index[0]
    dst = edge_index[1]

    b0 = b_msg0.reshape(1, H)
    b1 = b_msg1.reshape(1, H)
    bu0 = b_upd0.reshape(1, H)
    bu1 = b_upd1.reshape(1, H)

    c0 = _tc_cpre(edge_attr, W_msg0[2 * D:], b0)
    a0, bmat0 = _tc_pre(x, W_msg0[:D], W_msg0[D:2 * D])
    aggs0 = _sc_edge(a0, bmat0, c0, src, dst).reshape(NC, NP, H)
    c1 = _tc_cpre(edge_attr, W_msg1[2 * H:], b1)
    h, a1, bmat1 = _tc_update_fused(
        x, aggs0, W_upd0[:D], W_upd0[D:], bu0,
        W_msg1[:H], W_msg1[H:2 * H])

    aggs1 = _sc_edge(a1, bmat1, c1, src, dst).reshape(NC, NP, H)
    out = _tc_update(h, aggs1, W_upd1[:H], W_upd1[H:], bu1)
    return out


# speedup vs baseline: 1.0619x; 1.0121x over previous
import functools

import jax
import jax.numpy as jnp
from jax import lax
from jax.experimental import pallas as pl
from jax.experimental.pallas import tpu as pltpu
from jax.experimental.pallas import tpu_sc as plsc

N = 10000
E = 320000
D = 128
H = 128
ED = 16

NC = 2
NS = 16
NW = NC * NS
K = 40
EPT = E // NW
CHUNKS = EPT // K
IBLK = 25
NBLK = CHUNKS // IBLK
NP = 10112
RPT = NP // NS
LANES = 16
HW = H // 2

_DOT = functools.partial(
    lax.dot_general,
    dimension_numbers=(((1,), (0,)), ((), ())),
    preferred_element_type=jnp.float32,
    precision=lax.Precision.HIGHEST,
)


def _pack_bf16_pair(v):
    i = lax.bitcast_convert_type(v, jnp.int32)
    rbit = jnp.bitwise_and(lax.shift_right_logical(i, 16), 1)
    r = jnp.bitwise_and(
        lax.shift_right_arithmetic(i + 0x7FFF + rbit, 16), 0xFFFF)
    return jnp.bitwise_or(r[:, :HW], lax.shift_left(r[:, HW:], 16))


def _pre_body(x_ref, wi_ref, wj_ref, a_ref, b_ref):
    xb = x_ref[...]
    a_ref[...] = _DOT(xb, wi_ref[...])
    b_ref[...] = _DOT(xb, wj_ref[...])


def _cpre_body(ea_ref, we_ref, b_ref, c_ref):
    c_ref[...] = _DOT(ea_ref[...], we_ref[...]) + b_ref[...]


def _xu_body(x_ref, wux_ref, bu_ref, u_ref):
    u_ref[...] = _DOT(x_ref[...], wux_ref[...]) + bu_ref[...]


def _upd_fused_body(u_ref, a0_ref, a1_ref, wua_ref,
                    wi_ref, wj_ref, h_ref, a_ref, b_ref):
    agg = a0_ref[0] + a1_ref[0]
    h = jnp.maximum(u_ref[...] + _DOT(agg, wua_ref[...]), 0.0)
    h_ref[...] = h
    a_ref[...] = _DOT(h, wi_ref[...])
    b_ref[...] = _DOT(h, wj_ref[...])


def _upd_body(u_ref, a0_ref, a1_ref, wua_ref, h_ref):
    agg = a0_ref[0] + a1_ref[0]
    h_ref[...] = jnp.maximum(u_ref[...] + _DOT(agg, wua_ref[...]), 0.0)


_BN = 1000
_BE = 4000


def _node_spec(shape):
    return pl.BlockSpec((_BN,) + shape[1:], lambda i: (i,) + (0,) * (len(shape) - 1))


def _full_spec(shape):
    return pl.BlockSpec(shape, lambda i: (0,) * len(shape))


def _tc_pre(x, wi, wj):
    return pl.pallas_call(
        _pre_body,
        grid=(N // _BN,),
        in_specs=[_node_spec((N, D)), _full_spec((D, H)), _full_spec((D, H))],
        out_specs=[_node_spec((N, H)), _node_spec((N, H))],
        out_shape=[jax.ShapeDtypeStruct((N, H), jnp.float32)] * 2,
    )(x, wi, wj)


def _tc_cpre(ea, we, b):
    espec = pl.BlockSpec((_BE, ED), lambda i: (i, 0))
    ospec = pl.BlockSpec((_BE, H), lambda i: (i, 0))
    return pl.pallas_call(
        _cpre_body,
        grid=(E // _BE,),
        in_specs=[espec, _full_spec((ED, H)), _full_spec((1, H))],
        out_specs=ospec,
        out_shape=jax.ShapeDtypeStruct((E, H), jnp.float32),
    )(ea, we, b)


_A0SPEC = pl.BlockSpec((1, _BN, H), lambda i: (0, i, 0))
_A1SPEC = pl.BlockSpec((1, _BN, H), lambda i: (1, i, 0))


def _tc_xu(x, wux, bu):
    return pl.pallas_call(
        _xu_body,
        grid=(N // _BN,),
        in_specs=[_node_spec((N, H)), _full_spec((H, H)), _full_spec((1, H))],
        out_specs=_node_spec((N, H)),
        out_shape=jax.ShapeDtypeStruct((N, H), jnp.float32),
    )(x, wux, bu)


def _tc_update_fused(u, aggs, wua, wi, wj):
    return pl.pallas_call(
        _upd_fused_body,
        grid=(N // _BN,),
        in_specs=[_node_spec((N, H)), _A0SPEC, _A1SPEC,
                  _full_spec((H, H)), _full_spec((H, H)), _full_spec((H, H))],
        out_specs=[_node_spec((N, H))] * 3,
        out_shape=[jax.ShapeDtypeStruct((N, H), jnp.float32)] * 3,
    )(u, aggs, aggs, wua, wi, wj)


def _tc_update(u, aggs, wua):
    return pl.pallas_call(
        _upd_body,
        grid=(N // _BN,),
        in_specs=[_node_spec((N, H)), _A0SPEC, _A1SPEC, _full_spec((H, H))],
        out_specs=_node_spec((N, H)),
        out_shape=jax.ShapeDtypeStruct((N, H), jnp.float32),
    )(u, aggs, aggs, wua)


def _unpack_pair(w):
    lo = lax.bitcast_convert_type(lax.shift_left(w, 16), jnp.float32)
    hi = lax.bitcast_convert_type(
        jnp.bitwise_and(w, jnp.int32(-65536)), jnp.float32)
    return lo, hi


def _sc_edge_body(a_hbm, b_hbm, c_hbm, src_hbm, dst_hbm, out_hbm,
                  shared, idx_d0, idx_d1, idx_s0, idx_s1,
                  buf_a0, buf_a1, buf_b0, buf_b1, buf_c0, buf_c1,
                  sem_a0, sem_a1, sem_b0, sem_b1, sem_c0, sem_c1,
                  sem_s0, sem_s1, sem_id0, sem_id1, sem_is0, sem_is1):
    c = lax.axis_index("c")
    s = lax.axis_index("s")
    g = c * NS + s

    idx_d = (idx_d0, idx_d1)
    idx_s = (idx_s0, idx_s1)
    buf_a = (buf_a0, buf_a1)
    buf_b = (buf_b0, buf_b1)
    buf_c = (buf_c0, buf_c1)
    sem_a = (sem_a0, sem_a1)
    sem_b = (sem_b0, sem_b1)
    sem_c = (sem_c0, sem_c1)
    sem_s = (sem_s0, sem_s1)
    sem_id = (sem_id0, sem_id1)
    sem_is = (sem_is0, sem_is1)

    def _load_idx(b, q):
        pltpu.async_copy(dst_hbm.at[g, b], idx_d[q], sem_id[q])
        pltpu.async_copy(src_hbm.at[g, b], idx_s[q], sem_is[q])

    def _wait_idx(q):
        pltpu.make_async_copy(dst_hbm.at[0, 0], idx_d[q], sem_id[q]).wait()
        pltpu.make_async_copy(src_hbm.at[0, 0], idx_s[q], sem_is[q]).wait()

    _load_idx(0, 0)

    zero = jnp.zeros((LANES,), jnp.float32)

    def _zrow(r, _):
        for j in range(H // LANES):
            buf_a0[r, pl.ds(j * LANES, LANES)] = zero
        return 0
    lax.fori_loop(0, K, _zrow, 0, unroll=False)

    rbase = pl.multiple_of(s * RPT, 8)
    nfull = RPT // K
    rem = RPT - nfull * K
    for j in range(nfull):
        pltpu.sync_copy(buf_a0, shared.at[pl.ds(rbase + j * K, K)])
    if rem:
        pltpu.sync_copy(buf_a0.at[pl.ds(0, rem)],
                        shared.at[pl.ds(rbase + nfull * K, rem)])

    plsc.subcore_barrier()

    def _gather_ab(blk, j, p, q):
        pltpu.async_copy(a_hbm.at[idx_d[q].at[j]], buf_a[p], sem_a[p])
        pltpu.async_copy(b_hbm.at[idx_s[q].at[j]], buf_b[p], sem_b[p])

    def _load_c(blk, j, p):
        eoff = pl.multiple_of(g * EPT + (blk * IBLK + j) * K, 8)
        pltpu.async_copy(c_hbm.at[pl.ds(eoff, K)], buf_c[p], sem_c[p])

    def _step(blk, j, p, q):
        pltpu.make_async_copy(a_hbm.at[pl.ds(0, K)], buf_a[p], sem_a[p]).wait()
        pltpu.make_async_copy(b_hbm.at[pl.ds(0, K)], buf_b[p], sem_b[p]).wait()
        pltpu.make_async_copy(c_hbm.at[pl.ds(0, K)], buf_c[p], sem_c[p]).wait()

        def _row(r, _):
            for jj in range(H // LANES):
                sl = pl.ds(jj * LANES, LANES)
                v = buf_a[p][r, sl] + buf_b[p][r, sl] + buf_c[p][r, sl]
                buf_c[p][r, sl] = jnp.maximum(v, 0.0)
            return 0
        lax.fori_loop(0, K, _row, 0, unroll=False)

        pltpu.async_copy(buf_c[p], shared.at[idx_d[q].at[j]], sem_s[p],
                         add=True)
        nxt = j + 2

        @pl.when(nxt < IBLK)
        def _():
            _gather_ab(blk, nxt, p, q)

        pltpu.make_async_copy(buf_c[p], shared.at[pl.ds(0, K)], sem_s[p]).wait()

        @pl.when(nxt < IBLK)
        def _():
            _load_c(blk, nxt, p)

    for blk in range(NBLK):
        q = blk % 2
        _wait_idx(q)
        if blk + 1 < NBLK:
            _load_idx(blk + 1, 1 - q)
        _gather_ab(blk, 0, 0, q)
        _load_c(blk, 0, 0)
        _gather_ab(blk, 1, 1, q)
        _load_c(blk, 1, 1)

        def _super(t, _):
            _step(blk, 2 * t, 0, q)
            _step(blk, 2 * t + 1, 1, q)
            return 0
        lax.fori_loop(0, IBLK // 2, _super, 0, unroll=False)
        if IBLK % 2:
            _step(blk, IBLK - 1, 0, q)

    plsc.subcore_barrier()

    obase = pl.multiple_of(c * NP + rbase, 8)
    for j in range(nfull):
        pltpu.sync_copy(shared.at[pl.ds(rbase + j * K, K)], buf_a0)
        pltpu.sync_copy(buf_a0, out_hbm.at[pl.ds(obase + j * K, K)])
    if rem:
        pltpu.sync_copy(shared.at[pl.ds(rbase + nfull * K, rem)],
                        buf_a0.at[pl.ds(0, rem)])
        pltpu.sync_copy(buf_a0.at[pl.ds(0, rem)],
                        out_hbm.at[pl.ds(obase + nfull * K, rem)])


@functools.cache
def _sc_edge_kernel():
    return pl.kernel(
        _sc_edge_body,
        out_type=jax.ShapeDtypeStruct((NC * NP, H), jnp.float32),
        mesh=plsc.VectorSubcoreMesh(core_axis_name="c", subcore_axis_name="s"),
        scratch_types=[
            pltpu.VMEM_SHARED((NP, H), jnp.float32),
        ] + [pltpu.VMEM((IBLK, K), jnp.int32)] * 4
          + [pltpu.VMEM((K, H), jnp.float32)] * 6
          + [pltpu.SemaphoreType.DMA] * 12,
    )


def _sc_edge(a, b, c, src, dst):
    src4 = src.reshape(NW, NBLK, IBLK, K)
    dst4 = dst.reshape(NW, NBLK, IBLK, K)
    return _sc_edge_kernel()(a, b, c, src4, dst4)


@jax.jit
def kernel(x, edge_index, edge_attr, W_msg0, b_msg0, W_upd0, b_upd0,
           W_msg1, b_msg1, W_upd1, b_upd1):
    src = edge_index[0]
    dst = edge_index[1]

    b0 = b_msg0.reshape(1, H)
    b1 = b_msg1.reshape(1, H)
    bu0 = b_upd0.reshape(1, H)
    bu1 = b_upd1.reshape(1, H)

    c0 = _tc_cpre(edge_attr, W_msg0[2 * D:], b0)
    a0, bmat0 = _tc_pre(x, W_msg0[:D], W_msg0[D:2 * D])
    aggs0 = _sc_edge(a0, bmat0, c0, src, dst).reshape(NC, NP, H)
    c1 = _tc_cpre(edge_attr, W_msg1[2 * H:], b1)
    u0 = _tc_xu(x, W_upd0[:D], bu0)
    h, a1, bmat1 = _tc_update_fused(
        u0, aggs0, W_upd0[D:], W_msg1[:H], W_msg1[H:2 * H])

    aggs1 = _sc_edge(a1, bmat1, c1, src, dst).reshape(NC, NP, H)
    u1 = _tc_xu(h, W_upd1[:H], bu1)
    out = _tc_update(u1, aggs1, W_upd1[H:])
    return out


# speedup vs baseline: 1.0619x; 1.0000x over previous
import functools

import jax
import jax.numpy as jnp
from jax import lax
from jax.experimental import pallas as pl
from jax.experimental.pallas import tpu as pltpu
from jax.experimental.pallas import tpu_sc as plsc

N = 10000
E = 320000
D = 128
H = 128
ED = 16

NC = 2
NS = 16
NW = NC * NS
K = 40
EPT = E // NW
CHUNKS = EPT // K
IBLK = 25
NBLK = CHUNKS // IBLK
NP = 10112
RPT = NP // NS
LANES = 16

_DOT = functools.partial(
    lax.dot_general,
    dimension_numbers=(((1,), (0,)), ((), ())),
    preferred_element_type=jnp.float32,
    precision=lax.Precision.HIGHEST,
)


def _pre_body(x_ref, wi_ref, wj_ref, a_ref, b_ref):
    xb = x_ref[...]
    a_ref[...] = _DOT(xb, wi_ref[...])
    b_ref[...] = _DOT(xb, wj_ref[...])


def _cpre_body(ea_ref, we_ref, b_ref, c_ref):
    c_ref[...] = _DOT(ea_ref[...], we_ref[...]) + b_ref[...]


def _xu_body(x_ref, wux_ref, bu_ref, u_ref):
    u_ref[...] = _DOT(x_ref[...], wux_ref[...]) + bu_ref[...]


def _upd_fused_body(u_ref, a0_ref, a1_ref, wua_ref,
                    wi_ref, wj_ref, h_ref, a_ref, b_ref):
    agg = a0_ref[0] + a1_ref[0]
    h = jnp.maximum(u_ref[...] + _DOT(agg, wua_ref[...]), 0.0)
    h_ref[...] = h
    a_ref[...] = _DOT(h, wi_ref[...])
    b_ref[...] = _DOT(h, wj_ref[...])


def _upd_body(u_ref, a0_ref, a1_ref, wua_ref, h_ref):
    agg = a0_ref[0] + a1_ref[0]
    h_ref[...] = jnp.maximum(u_ref[...] + _DOT(agg, wua_ref[...]), 0.0)


_BN = 1000
_BE = 4000


def _node_spec(shape):
    return pl.BlockSpec((_BN,) + shape[1:], lambda i: (i,) + (0,) * (len(shape) - 1))


def _full_spec(shape):
    return pl.BlockSpec(shape, lambda i: (0,) * len(shape))


def _tc_pre(x, wi, wj):
    return pl.pallas_call(
        _pre_body,
        grid=(N // _BN,),
        in_specs=[_node_spec((N, D)), _full_spec((D, H)), _full_spec((D, H))],
        out_specs=[_node_spec((N, H)), _node_spec((N, H))],
        out_shape=[jax.ShapeDtypeStruct((N, H), jnp.float32)] * 2,
    )(x, wi, wj)


def _tc_cpre(ea, we, b):
    espec = pl.BlockSpec((_BE, ED), lambda i: (i, 0))
    ospec = pl.BlockSpec((_BE, H), lambda i: (i, 0))
    return pl.pallas_call(
        _cpre_body,
        grid=(E // _BE,),
        in_specs=[espec, _full_spec((ED, H)), _full_spec((1, H))],
        out_specs=ospec,
        out_shape=jax.ShapeDtypeStruct((E, H), jnp.float32),
    )(ea, we, b)


_A0SPEC = pl.BlockSpec((1, _BN, H), lambda i: (0, i, 0))
_A1SPEC = pl.BlockSpec((1, _BN, H), lambda i: (1, i, 0))


def _tc_xu(x, wux, bu):
    return pl.pallas_call(
        _xu_body,
        grid=(N // _BN,),
        in_specs=[_node_spec((N, H)), _full_spec((H, H)), _full_spec((1, H))],
        out_specs=_node_spec((N, H)),
        out_shape=jax.ShapeDtypeStruct((N, H), jnp.float32),
    )(x, wux, bu)


def _tc_update_fused(u, aggs, wua, wi, wj):
    return pl.pallas_call(
        _upd_fused_body,
        grid=(N // _BN,),
        in_specs=[_node_spec((N, H)), _A0SPEC, _A1SPEC,
                  _full_spec((H, H)), _full_spec((H, H)), _full_spec((H, H))],
        out_specs=[_node_spec((N, H))] * 3,
        out_shape=[jax.ShapeDtypeStruct((N, H), jnp.float32)] * 3,
    )(u, aggs, aggs, wua, wi, wj)


def _tc_update(u, aggs, wua):
    return pl.pallas_call(
        _upd_body,
        grid=(N // _BN,),
        in_specs=[_node_spec((N, H)), _A0SPEC, _A1SPEC, _full_spec((H, H))],
        out_specs=_node_spec((N, H)),
        out_shape=jax.ShapeDtypeStruct((N, H), jnp.float32),
    )(u, aggs, aggs, wua)


def _sc_edge_body(a_hbm, b_hbm, c_hbm, src_hbm, dst_hbm, out_hbm,
                  shared, idx_d0, idx_d1, idx_s0, idx_s1,
                  buf_a0, buf_a1, buf_b0, buf_b1, buf_c0, buf_c1,
                  sem_a0, sem_a1, sem_b0, sem_b1, sem_c0, sem_c1,
                  sem_s0, sem_s1, sem_id0, sem_id1, sem_is0, sem_is1):
    c = lax.axis_index("c")
    s = lax.axis_index("s")
    g = c * NS + s

    idx_d = (idx_d0, idx_d1)
    idx_s = (idx_s0, idx_s1)
    buf_a = (buf_a0, buf_a1)
    buf_b = (buf_b0, buf_b1)
    buf_c = (buf_c0, buf_c1)
    sem_a = (sem_a0, sem_a1)
    sem_b = (sem_b0, sem_b1)
    sem_c = (sem_c0, sem_c1)
    sem_s = (sem_s0, sem_s1)
    sem_id = (sem_id0, sem_id1)
    sem_is = (sem_is0, sem_is1)

    def _load_idx(b, q):
        pltpu.async_copy(dst_hbm.at[g, b], idx_d[q], sem_id[q])
        pltpu.async_copy(src_hbm.at[g, b], idx_s[q], sem_is[q])

    def _wait_idx(q):
        pltpu.make_async_copy(dst_hbm.at[0, 0], idx_d[q], sem_id[q]).wait()
        pltpu.make_async_copy(src_hbm.at[0, 0], idx_s[q], sem_is[q]).wait()

    _load_idx(0, 0)

    zero = jnp.zeros((LANES,), jnp.float32)

    def _zrow(r, _):
        for j in range(H // LANES):
            buf_a0[r, pl.ds(j * LANES, LANES)] = zero
        return 0
    lax.fori_loop(0, K, _zrow, 0, unroll=False)

    rbase = pl.multiple_of(s * RPT, 8)
    nfull = RPT // K
    rem = RPT - nfull * K
    for j in range(nfull):
        pltpu.sync_copy(buf_a0, shared.at[pl.ds(rbase + j * K, K)])
    if rem:
        pltpu.sync_copy(buf_a0.at[pl.ds(0, rem)],
                        shared.at[pl.ds(rbase + nfull * K, rem)])

    plsc.subcore_barrier()

    def _gather_ab(blk, j, p, q):
        pltpu.async_copy(a_hbm.at[idx_d[q].at[j]], buf_a[p], sem_a[p])
        pltpu.async_copy(b_hbm.at[idx_s[q].at[j]], buf_b[p], sem_b[p])

    def _load_c(blk, j, p):
        eoff = pl.multiple_of(g * EPT + (blk * IBLK + j) * K, 8)
        pltpu.async_copy(c_hbm.at[pl.ds(eoff, K)], buf_c[p], sem_c[p])

    def _step(blk, j, p, q):
        pltpu.make_async_copy(a_hbm.at[pl.ds(0, K)], buf_a[p], sem_a[p]).wait()
        pltpu.make_async_copy(b_hbm.at[pl.ds(0, K)], buf_b[p], sem_b[p]).wait()
        pltpu.make_async_copy(c_hbm.at[pl.ds(0, K)], buf_c[p], sem_c[p]).wait()

        def _row(r, _):
            for jj in range(H // LANES):
                sl = pl.ds(jj * LANES, LANES)
                v = buf_a[p][r, sl] + buf_b[p][r, sl] + buf_c[p][r, sl]
                buf_c[p][r, sl] = jnp.maximum(v, 0.0)
            return 0
        lax.fori_loop(0, K, _row, 0, unroll=False)

        pltpu.async_copy(buf_c[p], shared.at[idx_d[q].at[j]], sem_s[p],
                         add=True)
        nxt = j + 2

        @pl.when(nxt < IBLK)
        def _():
            _gather_ab(blk, nxt, p, q)

        pltpu.make_async_copy(buf_c[p], shared.at[pl.ds(0, K)], sem_s[p]).wait()

        @pl.when(nxt < IBLK)
        def _():
            _load_c(blk, nxt, p)

    for blk in range(NBLK):
        q = blk % 2
        _wait_idx(q)
        if blk + 1 < NBLK:
            _load_idx(blk + 1, 1 - q)
        _gather_ab(blk, 0, 0, q)
        _load_c(blk, 0, 0)
        _gather_ab(blk, 1, 1, q)
        _load_c(blk, 1, 1)

        def _super(t, _):
            _step(blk, 2 * t, 0, q)
            _step(blk, 2 * t + 1, 1, q)
            return 0
        lax.fori_loop(0, IBLK // 2, _super, 0, unroll=False)
        if IBLK % 2:
            _step(blk, IBLK - 1, 0, q)

    plsc.subcore_barrier()

    obase = pl.multiple_of(c * NP + rbase, 8)
    for j in range(nfull):
        pltpu.sync_copy(shared.at[pl.ds(rbase + j * K, K)], buf_a0)
        pltpu.sync_copy(buf_a0, out_hbm.at[pl.ds(obase + j * K, K)])
    if rem:
        pltpu.sync_copy(shared.at[pl.ds(rbase + nfull * K, rem)],
                        buf_a0.at[pl.ds(0, rem)])
        pltpu.sync_copy(buf_a0.at[pl.ds(0, rem)],
                        out_hbm.at[pl.ds(obase + nfull * K, rem)])


@functools.cache
def _sc_edge_kernel():
    return pl.kernel(
        _sc_edge_body,
        out_type=jax.ShapeDtypeStruct((NC * NP, H), jnp.float32),
        mesh=plsc.VectorSubcoreMesh(core_axis_name="c", subcore_axis_name="s"),
        scratch_types=[
            pltpu.VMEM_SHARED((NP, H), jnp.float32),
        ] + [pltpu.VMEM((IBLK, K), jnp.int32)] * 4
          + [pltpu.VMEM((K, H), jnp.float32)] * 6
          + [pltpu.SemaphoreType.DMA] * 12,
    )


def _sc_edge(a, b, c, src, dst):
    src4 = src.reshape(NW, NBLK, IBLK, K)
    dst4 = dst.reshape(NW, NBLK, IBLK, K)
    return _sc_edge_kernel()(a, b, c, src4, dst4)


@jax.jit
def kernel(x, edge_index, edge_attr, W_msg0, b_msg0, W_upd0, b_upd0,
           W_msg1, b_msg1, W_upd1, b_upd1):
    src = edge_index[0]
    dst = edge_index[1]

    b0 = b_msg0.reshape(1, H)
    b1 = b_msg1.reshape(1, H)
    bu0 = b_upd0.reshape(1, H)
    bu1 = b_upd1.reshape(1, H)

    c0 = _tc_cpre(edge_attr, W_msg0[2 * D:], b0)
    a0, bmat0 = _tc_pre(x, W_msg0[:D], W_msg0[D:2 * D])
    aggs0 = _sc_edge(a0, bmat0, c0, src, dst).reshape(NC, NP, H)
    c1 = _tc_cpre(edge_attr, W_msg1[2 * H:], b1)
    u0 = _tc_xu(x, W_upd0[:D], bu0)
    h, a1, bmat1 = _tc_update_fused(
        u0, aggs0, W_upd0[D:], W_msg1[:H], W_msg1[H:2 * H])

    aggs1 = _sc_edge(a1, bmat1, c1, src, dst).reshape(NC, NP, H)
    u1 = _tc_xu(h, W_upd1[:H], bu1)
    out = _tc_update(u1, aggs1, W_upd1[H:])
    return out


# speedup vs baseline: 1.0770x; 1.0143x over previous
import functools

import jax
import jax.numpy as jnp
from jax import lax
from jax.experimental import pallas as pl
from jax.experimental.pallas import tpu as pltpu
from jax.experimental.pallas import tpu_sc as plsc

N = 10000
E = 320000
D = 128
H = 128
ED = 16

NC = 2
NS = 16
NW = NC * NS
K = 40
EPT = E // NW
CHUNKS = EPT // K
IBLK = 25
NBLK = CHUNKS // IBLK
NP = 10112
RPT = NP // NS
LANES = 16

_DOT = functools.partial(
    lax.dot_general,
    dimension_numbers=(((1,), (0,)), ((), ())),
    preferred_element_type=jnp.float32,
    precision=lax.Precision.HIGHEST,
)


def _pre_body(x_ref, wi_ref, wj_ref, a_ref, b_ref):
    xb = x_ref[...]
    a_ref[...] = _DOT(xb, wi_ref[...])
    b_ref[...] = _DOT(xb, wj_ref[...])


def _cpre_body(ea_ref, we_ref, b_ref, c_ref):
    c_ref[...] = _DOT(ea_ref[...], we_ref[...]) + b_ref[...]


def _xu_body(x_ref, wux_ref, bu_ref, u_ref):
    u_ref[...] = _DOT(x_ref[...], wux_ref[...]) + bu_ref[...]


def _upd_fused_body(u_ref, a0_ref, a1_ref, wua_ref,
                    wi_ref, wj_ref, h_ref, a_ref, b_ref):
    agg = a0_ref[0] + a1_ref[0]
    h = jnp.maximum(u_ref[...] + _DOT(agg, wua_ref[...]), 0.0)
    h_ref[...] = h
    a_ref[...] = _DOT(h, wi_ref[...])
    b_ref[...] = _DOT(h, wj_ref[...])


def _upd_body(u_ref, a0_ref, a1_ref, wua_ref, h_ref):
    agg = a0_ref[0] + a1_ref[0]
    h_ref[...] = jnp.maximum(u_ref[...] + _DOT(agg, wua_ref[...]), 0.0)


_BN = 1000
_BE = 8000


def _node_spec(shape):
    return pl.BlockSpec((_BN,) + shape[1:], lambda i: (i,) + (0,) * (len(shape) - 1))


def _full_spec(shape):
    return pl.BlockSpec(shape, lambda i: (0,) * len(shape))


def _tc_pre(x, wi, wj):
    return pl.pallas_call(
        _pre_body,
        grid=(N // _BN,),
        in_specs=[_node_spec((N, D)), _full_spec((D, H)), _full_spec((D, H))],
        out_specs=[_node_spec((N, H)), _node_spec((N, H))],
        out_shape=[jax.ShapeDtypeStruct((N, H), jnp.float32)] * 2,
    )(x, wi, wj)


def _tc_cpre(ea, we, b):
    espec = pl.BlockSpec((_BE, ED), lambda i: (i, 0))
    ospec = pl.BlockSpec((_BE, H), lambda i: (i, 0))
    return pl.pallas_call(
        _cpre_body,
        grid=(E // _BE,),
        in_specs=[espec, _full_spec((ED, H)), _full_spec((1, H))],
        out_specs=ospec,
        out_shape=jax.ShapeDtypeStruct((E, H), jnp.float32),
    )(ea, we, b)


_A0SPEC = pl.BlockSpec((1, _BN, H), lambda i: (0, i, 0))
_A1SPEC = pl.BlockSpec((1, _BN, H), lambda i: (1, i, 0))


def _tc_xu(x, wux, bu):
    return pl.pallas_call(
        _xu_body,
        grid=(N // _BN,),
        in_specs=[_node_spec((N, H)), _full_spec((H, H)), _full_spec((1, H))],
        out_specs=_node_spec((N, H)),
        out_shape=jax.ShapeDtypeStruct((N, H), jnp.float32),
    )(x, wux, bu)


def _tc_update_fused(u, aggs, wua, wi, wj):
    return pl.pallas_call(
        _upd_fused_body,
        grid=(N // _BN,),
        in_specs=[_node_spec((N, H)), _A0SPEC, _A1SPEC,
                  _full_spec((H, H)), _full_spec((H, H)), _full_spec((H, H))],
        out_specs=[_node_spec((N, H))] * 3,
        out_shape=[jax.ShapeDtypeStruct((N, H), jnp.float32)] * 3,
    )(u, aggs, aggs, wua, wi, wj)


def _tc_update(u, aggs, wua):
    return pl.pallas_call(
        _upd_body,
        grid=(N // _BN,),
        in_specs=[_node_spec((N, H)), _A0SPEC, _A1SPEC, _full_spec((H, H))],
        out_specs=_node_spec((N, H)),
        out_shape=jax.ShapeDtypeStruct((N, H), jnp.float32),
    )(u, aggs, aggs, wua)


def _sc_edge_body(a_hbm, b_hbm, c_hbm, src_hbm, dst_hbm, out_hbm,
                  shared, idx_d0, idx_d1, idx_s0, idx_s1,
                  buf_a0, buf_a1, buf_b0, buf_b1, buf_c0, buf_c1,
                  sem_a0, sem_a1, sem_b0, sem_b1, sem_c0, sem_c1,
                  sem_s0, sem_s1, sem_id0, sem_id1, sem_is0, sem_is1):
    c = lax.axis_index("c")
    s = lax.axis_index("s")
    g = c * NS + s

    idx_d = (idx_d0, idx_d1)
    idx_s = (idx_s0, idx_s1)
    buf_a = (buf_a0, buf_a1)
    buf_b = (buf_b0, buf_b1)
    buf_c = (buf_c0, buf_c1)
    sem_a = (sem_a0, sem_a1)
    sem_b = (sem_b0, sem_b1)
    sem_c = (sem_c0, sem_c1)
    sem_s = (sem_s0, sem_s1)
    sem_id = (sem_id0, sem_id1)
    sem_is = (sem_is0, sem_is1)

    def _load_idx(b, q):
        pltpu.async_copy(dst_hbm.at[g, b], idx_d[q], sem_id[q])
        pltpu.async_copy(src_hbm.at[g, b], idx_s[q], sem_is[q])

    def _wait_idx(q):
        pltpu.make_async_copy(dst_hbm.at[0, 0], idx_d[q], sem_id[q]).wait()
        pltpu.make_async_copy(src_hbm.at[0, 0], idx_s[q], sem_is[q]).wait()

    _load_idx(0, 0)

    zero = jnp.zeros((LANES,), jnp.float32)

    def _zrow(r, _):
        for j in range(H // LANES):
            buf_a0[r, pl.ds(j * LANES, LANES)] = zero
        return 0
    lax.fori_loop(0, K, _zrow, 0, unroll=False)

    rbase = pl.multiple_of(s * RPT, 8)
    nfull = RPT // K
    rem = RPT - nfull * K
    for j in range(nfull):
        pltpu.sync_copy(buf_a0, shared.at[pl.ds(rbase + j * K, K)])
    if rem:
        pltpu.sync_copy(buf_a0.at[pl.ds(0, rem)],
                        shared.at[pl.ds(rbase + nfull * K, rem)])

    plsc.subcore_barrier()

    def _gather_ab(blk, j, p, q):
        pltpu.async_copy(a_hbm.at[idx_d[q].at[j]], buf_a[p], sem_a[p])
        pltpu.async_copy(b_hbm.at[idx_s[q].at[j]], buf_b[p], sem_b[p])

    def _load_c(blk, j, p):
        eoff = pl.multiple_of(g * EPT + (blk * IBLK + j) * K, 8)
        pltpu.async_copy(c_hbm.at[pl.ds(eoff, K)], buf_c[p], sem_c[p])

    def _step(blk, j, p, q):
        pltpu.make_async_copy(a_hbm.at[pl.ds(0, K)], buf_a[p], sem_a[p]).wait()
        pltpu.make_async_copy(b_hbm.at[pl.ds(0, K)], buf_b[p], sem_b[p]).wait()
        pltpu.make_async_copy(c_hbm.at[pl.ds(0, K)], buf_c[p], sem_c[p]).wait()

        def _row(r, _):
            for jj in range(H // LANES):
                sl = pl.ds(jj * LANES, LANES)
                v = buf_a[p][r, sl] + buf_b[p][r, sl] + buf_c[p][r, sl]
                buf_c[p][r, sl] = jnp.maximum(v, 0.0)
            return 0
        lax.fori_loop(0, K, _row, 0, unroll=False)

        pltpu.async_copy(buf_c[p], shared.at[idx_d[q].at[j]], sem_s[p],
                         add=True)
        nxt = j + 2

        @pl.when(nxt < IBLK)
        def _():
            _gather_ab(blk, nxt, p, q)

        pltpu.make_async_copy(buf_c[p], shared.at[pl.ds(0, K)], sem_s[p]).wait()

        @pl.when(nxt < IBLK)
        def _():
            _load_c(blk, nxt, p)

    for blk in range(NBLK):
        q = blk % 2
        _wait_idx(q)
        if blk + 1 < NBLK:
            _load_idx(blk + 1, 1 - q)
        _gather_ab(blk, 0, 0, q)
        _load_c(blk, 0, 0)
        _gather_ab(blk, 1, 1, q)
        _load_c(blk, 1, 1)

        def _super(t, _):
            _step(blk, 2 * t, 0, q)
            _step(blk, 2 * t + 1, 1, q)
            return 0
        lax.fori_loop(0, IBLK // 2, _super, 0, unroll=False)
        if IBLK % 2:
            _step(blk, IBLK - 1, 0, q)

    plsc.subcore_barrier()

    obase = pl.multiple_of(c * NP + rbase, 8)
    for j in range(nfull):
        pltpu.sync_copy(shared.at[pl.ds(rbase + j * K, K)], buf_a0)
        pltpu.sync_copy(buf_a0, out_hbm.at[pl.ds(obase + j * K, K)])
    if rem:
        pltpu.sync_copy(shared.at[pl.ds(rbase + nfull * K, rem)],
                        buf_a0.at[pl.ds(0, rem)])
        pltpu.sync_copy(buf_a0.at[pl.ds(0, rem)],
                        out_hbm.at[pl.ds(obase + nfull * K, rem)])


@functools.cache
def _sc_edge_kernel():
    return pl.kernel(
        _sc_edge_body,
        out_type=jax.ShapeDtypeStruct((NC * NP, H), jnp.float32),
        mesh=plsc.VectorSubcoreMesh(core_axis_name="c", subcore_axis_name="s"),
        scratch_types=[
            pltpu.VMEM_SHARED((NP, H), jnp.float32),
        ] + [pltpu.VMEM((IBLK, K), jnp.int32)] * 4
          + [pltpu.VMEM((K, H), jnp.float32)] * 6
          + [pltpu.SemaphoreType.DMA] * 12,
    )


def _sc_edge(a, b, c, src, dst):
    src4 = src.reshape(NW, NBLK, IBLK, K)
    dst4 = dst.reshape(NW, NBLK, IBLK, K)
    return _sc_edge_kernel()(a, b, c, src4, dst4)


@jax.jit
def kernel(x, edge_index, edge_attr, W_msg0, b_msg0, W_upd0, b_upd0,
           W_msg1, b_msg1, W_upd1, b_upd1):
    src = edge_index[0]
    dst = edge_index[1]

    b0 = b_msg0.reshape(1, H)
    b1 = b_msg1.reshape(1, H)
    bu0 = b_upd0.reshape(1, H)
    bu1 = b_upd1.reshape(1, H)

    c0 = _tc_cpre(edge_attr, W_msg0[2 * D:], b0)
    a0, bmat0 = _tc_pre(x, W_msg0[:D], W_msg0[D:2 * D])
    aggs0 = _sc_edge(a0, bmat0, c0, src, dst).reshape(NC, NP, H)
    c1 = _tc_cpre(edge_attr, W_msg1[2 * H:], b1)
    u0 = _tc_xu(x, W_upd0[:D], bu0)
    h, a1, bmat1 = _tc_update_fused(
        u0, aggs0, W_upd0[D:], W_msg1[:H], W_msg1[H:2 * H])

    aggs1 = _sc_edge(a1, bmat1, c1, src, dst).reshape(NC, NP, H)
    u1 = _tc_xu(h, W_upd1[:H], bu1)
    out = _tc_update(u1, aggs1, W_upd1[H:])
    return out


# speedup vs baseline: 1.1056x; 1.0265x over previous
import functools

import jax
import jax.numpy as jnp
from jax import lax
from jax.experimental import pallas as pl
from jax.experimental.pallas import tpu as pltpu
from jax.experimental.pallas import tpu_sc as plsc

N = 10000
E = 320000
D = 128
H = 128
ED = 16

NC = 2
NS = 16
NW = NC * NS
K = 40
EPT = E // NW
CHUNKS = EPT // K
IBLK = 25
NBLK = CHUNKS // IBLK
NP = 10112
RPT = NP // NS
LANES = 16

_DOT = functools.partial(
    lax.dot_general,
    dimension_numbers=(((1,), (0,)), ((), ())),
    preferred_element_type=jnp.float32,
    precision=lax.Precision.HIGHEST,
)


def _pre_body(x_ref, wi_ref, wj_ref, a_ref, b_ref):
    xb = x_ref[...]
    a_ref[...] = _DOT(xb, wi_ref[...])
    b_ref[...] = _DOT(xb, wj_ref[...])


def _cpre_body(ea_ref, we_ref, b_ref, c_ref):
    c_ref[...] = _DOT(ea_ref[...], we_ref[...]) + b_ref[...]


def _xu_body(x_ref, wux_ref, bu_ref, u_ref):
    u_ref[...] = _DOT(x_ref[...], wux_ref[...]) + bu_ref[...]


def _upd_fused_body(u_ref, a0_ref, a1_ref, wua_ref,
                    wi_ref, wj_ref, h_ref, a_ref, b_ref):
    agg = a0_ref[0] + a1_ref[0]
    h = jnp.maximum(u_ref[...] + _DOT(agg, wua_ref[...]), 0.0)
    h_ref[...] = h
    a_ref[...] = _DOT(h, wi_ref[...])
    b_ref[...] = _DOT(h, wj_ref[...])


def _upd_body(u_ref, a0_ref, a1_ref, wua_ref, h_ref):
    agg = a0_ref[0] + a1_ref[0]
    h_ref[...] = jnp.maximum(u_ref[...] + _DOT(agg, wua_ref[...]), 0.0)


_BN = 2000
_BE = 8000


def _node_spec(shape):
    return pl.BlockSpec((_BN,) + shape[1:], lambda i: (i,) + (0,) * (len(shape) - 1))


def _full_spec(shape):
    return pl.BlockSpec(shape, lambda i: (0,) * len(shape))


def _tc_pre(x, wi, wj):
    return pl.pallas_call(
        _pre_body,
        grid=(N // _BN,),
        in_specs=[_node_spec((N, D)), _full_spec((D, H)), _full_spec((D, H))],
        out_specs=[_node_spec((N, H)), _node_spec((N, H))],
        out_shape=[jax.ShapeDtypeStruct((N, H), jnp.float32)] * 2,
    )(x, wi, wj)


def _tc_cpre(ea, we, b):
    espec = pl.BlockSpec((_BE, ED), lambda i: (i, 0))
    ospec = pl.BlockSpec((_BE, H), lambda i: (i, 0))
    return pl.pallas_call(
        _cpre_body,
        grid=(E // _BE,),
        in_specs=[espec, _full_spec((ED, H)), _full_spec((1, H))],
        out_specs=ospec,
        out_shape=jax.ShapeDtypeStruct((E, H), jnp.float32),
    )(ea, we, b)


_A0SPEC = pl.BlockSpec((1, _BN, H), lambda i: (0, i, 0))
_A1SPEC = pl.BlockSpec((1, _BN, H), lambda i: (1, i, 0))


def _tc_xu(x, wux, bu):
    return pl.pallas_call(
        _xu_body,
        grid=(N // _BN,),
        in_specs=[_node_spec((N, H)), _full_spec((H, H)), _full_spec((1, H))],
        out_specs=_node_spec((N, H)),
        out_shape=jax.ShapeDtypeStruct((N, H), jnp.float32),
    )(x, wux, bu)


def _tc_update_fused(u, aggs, wua, wi, wj):
    return pl.pallas_call(
        _upd_fused_body,
        grid=(N // _BN,),
        in_specs=[_node_spec((N, H)), _A0SPEC, _A1SPEC,
                  _full_spec((H, H)), _full_spec((H, H)), _full_spec((H, H))],
        out_specs=[_node_spec((N, H))] * 3,
        out_shape=[jax.ShapeDtypeStruct((N, H), jnp.float32)] * 3,
    )(u, aggs, aggs, wua, wi, wj)


def _tc_update(u, aggs, wua):
    return pl.pallas_call(
        _upd_body,
        grid=(N // _BN,),
        in_specs=[_node_spec((N, H)), _A0SPEC, _A1SPEC, _full_spec((H, H))],
        out_specs=_node_spec((N, H)),
        out_shape=jax.ShapeDtypeStruct((N, H), jnp.float32),
    )(u, aggs, aggs, wua)


def _sc_edge_body(a_hbm, b_hbm, c_hbm, src_hbm, dst_hbm, out_hbm,
                  shared, idx_d0, idx_d1, idx_s0, idx_s1,
                  buf_a0, buf_a1, buf_b0, buf_b1, buf_c0, buf_c1,
                  sem_a0, sem_a1, sem_b0, sem_b1, sem_c0, sem_c1,
                  sem_s0, sem_s1, sem_id0, sem_id1, sem_is0, sem_is1):
    c = lax.axis_index("c")
    s = lax.axis_index("s")
    g = c * NS + s

    idx_d = (idx_d0, idx_d1)
    idx_s = (idx_s0, idx_s1)
    buf_a = (buf_a0, buf_a1)
    buf_b = (buf_b0, buf_b1)
    buf_c = (buf_c0, buf_c1)
    sem_a = (sem_a0, sem_a1)
    sem_b = (sem_b0, sem_b1)
    sem_c = (sem_c0, sem_c1)
    sem_s = (sem_s0, sem_s1)
    sem_id = (sem_id0, sem_id1)
    sem_is = (sem_is0, sem_is1)

    def _load_idx(b, q):
        pltpu.async_copy(dst_hbm.at[g, b], idx_d[q], sem_id[q])
        pltpu.async_copy(src_hbm.at[g, b], idx_s[q], sem_is[q])

    def _wait_idx(q):
        pltpu.make_async_copy(dst_hbm.at[0, 0], idx_d[q], sem_id[q]).wait()
        pltpu.make_async_copy(src_hbm.at[0, 0], idx_s[q], sem_is[q]).wait()

    _load_idx(0, 0)

    zero = jnp.zeros((LANES,), jnp.float32)

    def _zrow(r, _):
        for j in range(H // LANES):
            buf_a0[r, pl.ds(j * LANES, LANES)] = zero
        return 0
    lax.fori_loop(0, K, _zrow, 0, unroll=False)

    rbase = pl.multiple_of(s * RPT, 8)
    nfull = RPT // K
    rem = RPT - nfull * K
    for j in range(nfull):
        pltpu.sync_copy(buf_a0, shared.at[pl.ds(rbase + j * K, K)])
    if rem:
        pltpu.sync_copy(buf_a0.at[pl.ds(0, rem)],
                        shared.at[pl.ds(rbase + nfull * K, rem)])

    plsc.subcore_barrier()

    def _gather_ab(blk, j, p, q):
        pltpu.async_copy(a_hbm.at[idx_d[q].at[j]], buf_a[p], sem_a[p])
        pltpu.async_copy(b_hbm.at[idx_s[q].at[j]], buf_b[p], sem_b[p])

    def _load_c(blk, j, p):
        eoff = pl.multiple_of(g * EPT + (blk * IBLK + j) * K, 8)
        pltpu.async_copy(c_hbm.at[pl.ds(eoff, K)], buf_c[p], sem_c[p])

    def _step(blk, j, p, q):
        pltpu.make_async_copy(a_hbm.at[pl.ds(0, K)], buf_a[p], sem_a[p]).wait()
        pltpu.make_async_copy(b_hbm.at[pl.ds(0, K)], buf_b[p], sem_b[p]).wait()
        pltpu.make_async_copy(c_hbm.at[pl.ds(0, K)], buf_c[p], sem_c[p]).wait()

        def _row(r, _):
            for jj in range(H // LANES):
                sl = pl.ds(jj * LANES, LANES)
                v = buf_a[p][r, sl] + buf_b[p][r, sl] + buf_c[p][r, sl]
                buf_c[p][r, sl] = jnp.maximum(v, 0.0)
            return 0
        lax.fori_loop(0, K, _row, 0, unroll=False)

        pltpu.async_copy(buf_c[p], shared.at[idx_d[q].at[j]], sem_s[p],
                         add=True)
        nxt = j + 2

        @pl.when(nxt < IBLK)
        def _():
            _gather_ab(blk, nxt, p, q)

        pltpu.make_async_copy(buf_c[p], shared.at[pl.ds(0, K)], sem_s[p]).wait()

        @pl.when(nxt < IBLK)
        def _():
            _load_c(blk, nxt, p)

    for blk in range(NBLK):
        q = blk % 2
        _wait_idx(q)
        if blk + 1 < NBLK:
            _load_idx(blk + 1, 1 - q)
        _gather_ab(blk, 0, 0, q)
        _load_c(blk, 0, 0)
        _gather_ab(blk, 1, 1, q)
        _load_c(blk, 1, 1)

        def _super(t, _):
            _step(blk, 2 * t, 0, q)
            _step(blk, 2 * t + 1, 1, q)
            return 0
        lax.fori_loop(0, IBLK // 2, _super, 0, unroll=False)
        if IBLK % 2:
            _step(blk, IBLK - 1, 0, q)

    plsc.subcore_barrier()

    obase = pl.multiple_of(c * NP + rbase, 8)
    for j in range(nfull):
        pltpu.sync_copy(shared.at[pl.ds(rbase + j * K, K)], buf_a0)
        pltpu.sync_copy(buf_a0, out_hbm.at[pl.ds(obase + j * K, K)])
    if rem:
        pltpu.sync_copy(shared.at[pl.ds(rbase + nfull * K, rem)],
                        buf_a0.at[pl.ds(0, rem)])
        pltpu.sync_copy(buf_a0.at[pl.ds(0, rem)],
                        out_hbm.at[pl.ds(obase + nfull * K, rem)])


@functools.cache
def _sc_edge_kernel():
    return pl.kernel(
        _sc_edge_body,
        out_type=jax.ShapeDtypeStruct((NC * NP, H), jnp.float32),
        mesh=plsc.VectorSubcoreMesh(core_axis_name="c", subcore_axis_name="s"),
        scratch_types=[
            pltpu.VMEM_SHARED((NP, H), jnp.float32),
        ] + [pltpu.VMEM((IBLK, K), jnp.int32)] * 4
          + [pltpu.VMEM((K, H), jnp.float32)] * 6
          + [pltpu.SemaphoreType.DMA] * 12,
    )


def _sc_edge(a, b, c, src, dst):
    src4 = src.reshape(NW, NBLK, IBLK, K)
    dst4 = dst.reshape(NW, NBLK, IBLK, K)
    return _sc_edge_kernel()(a, b, c, src4, dst4)


@jax.jit
def kernel(x, edge_index, edge_attr, W_msg0, b_msg0, W_upd0, b_upd0,
           W_msg1, b_msg1, W_upd1, b_upd1):
    src = edge_index[0]
    dst = edge_index[1]

    b0 = b_msg0.reshape(1, H)
    b1 = b_msg1.reshape(1, H)
    bu0 = b_upd0.reshape(1, H)
    bu1 = b_upd1.reshape(1, H)

    c0 = _tc_cpre(edge_attr, W_msg0[2 * D:], b0)
    a0, bmat0 = _tc_pre(x, W_msg0[:D], W_msg0[D:2 * D])
    aggs0 = _sc_edge(a0, bmat0, c0, src, dst).reshape(NC, NP, H)
    c1 = _tc_cpre(edge_attr, W_msg1[2 * H:], b1)
    u0 = _tc_xu(x, W_upd0[:D], bu0)
    h, a1, bmat1 = _tc_update_fused(
        u0, aggs0, W_upd0[D:], W_msg1[:H], W_msg1[H:2 * H])

    aggs1 = _sc_edge(a1, bmat1, c1, src, dst).reshape(NC, NP, H)
    u1 = _tc_xu(h, W_upd1[:H], bu1)
    out = _tc_update(u1, aggs1, W_upd1[H:])
    return out
